# speedup vs baseline: 6.4910x; 6.4910x over previous
import functools

import jax
import jax.numpy as jnp
from jax.experimental import pallas as pl
from jax.experimental.pallas import tpu as pltpu
from jax.experimental.pallas import tpu_sc as plsc

EMBED_DIM = 128
WINDOW = 128


def _gather_rows(weight, ids_2d, num_ids):
    mesh = plsc.VectorSubcoreMesh(
        core_axis_name="core", subcore_axis_name="subcore"
    )

    @functools.partial(
        pl.kernel,
        out_type=jax.ShapeDtypeStruct((num_ids, EMBED_DIM), jnp.float32),
        mesh=mesh,
    )
    def gather_kernel(w_hbm, i_hbm, o_hbm):
        def body(i_vmem, o_vmem):
            pltpu.sync_copy(w_hbm.at[i_vmem.at[0]], o_vmem)

        pltpu.emit_pipeline(
            body,
            grid=(num_ids // WINDOW,),
            in_specs=[
                pl.BlockSpec((1, WINDOW), index_map=lambda i: (0, i))
            ],
            out_specs=[
                pl.BlockSpec((WINDOW, EMBED_DIM), index_map=lambda i: (i, 0))
            ],
            core_axis_name=("core", "subcore"),
            dimension_semantics=(pltpu.PARALLEL,),
        )(i_hbm, o_hbm)

    return gather_kernel(weight, ids_2d)


def kernel(input_ids, weight):
    batch, seq = input_ids.shape
    num_ids = batch * seq
    ids_2d = input_ids.reshape(1, num_ids).astype(jnp.int32)
    out = _gather_rows(weight, ids_2d, num_ids)
    return out.reshape(batch, seq, EMBED_DIM)


# speedup vs baseline: 6.9031x; 1.0635x over previous
import functools

import jax
import jax.numpy as jnp
from jax.experimental import pallas as pl
from jax.experimental.pallas import tpu as pltpu
from jax.experimental.pallas import tpu_sc as plsc

EMBED_DIM = 128
WINDOW = 128
GROUP = 2


def _gather_rows(weight, ids_2d, num_ids):
    mesh = plsc.VectorSubcoreMesh(
        core_axis_name="core", subcore_axis_name="subcore"
    )

    @functools.partial(
        pl.kernel,
        out_type=jax.ShapeDtypeStruct((num_ids, EMBED_DIM), jnp.float32),
        mesh=mesh,
    )
    def gather_kernel(w_hbm, i_hbm, o_hbm):
        def body(i_vmem, o_vmem):
            for g in range(GROUP):
                pltpu.sync_copy(
                    w_hbm.at[i_vmem.at[g]],
                    o_vmem.at[pl.ds(g * WINDOW, WINDOW)],
                )

        pltpu.emit_pipeline(
            body,
            grid=(num_ids // (GROUP * WINDOW),),
            in_specs=[
                pl.BlockSpec((GROUP, WINDOW), index_map=lambda i: (i, 0))
            ],
            out_specs=[
                pl.BlockSpec(
                    (GROUP * WINDOW, EMBED_DIM), index_map=lambda i: (i, 0)
                )
            ],
            core_axis_name=("core", "subcore"),
            dimension_semantics=(pltpu.PARALLEL,),
        )(i_hbm, o_hbm)

    return gather_kernel(weight, ids_2d)


def kernel(input_ids, weight):
    batch, seq = input_ids.shape
    num_ids = batch * seq
    ids_2d = input_ids.reshape(num_ids // WINDOW, WINDOW).astype(jnp.int32)
    out = _gather_rows(weight, ids_2d, num_ids)
    return out.reshape(batch, seq, EMBED_DIM)


# speedup vs baseline: 7.7485x; 1.1225x over previous
import functools

import jax
import jax.numpy as jnp
from jax.experimental import pallas as pl
from jax.experimental.pallas import tpu as pltpu
from jax.experimental.pallas import tpu_sc as plsc

EMBED_DIM = 128
WINDOW = 256
GROUP = 1


def _gather_rows(weight, ids_2d, num_ids):
    mesh = plsc.VectorSubcoreMesh(
        core_axis_name="core", subcore_axis_name="subcore"
    )

    @functools.partial(
        pl.kernel,
        out_type=jax.ShapeDtypeStruct((num_ids, EMBED_DIM), jnp.float32),
        mesh=mesh,
    )
    def gather_kernel(w_hbm, i_hbm, o_hbm):
        def body(i_vmem, o_vmem):
            for g in range(GROUP):
                pltpu.sync_copy(
                    w_hbm.at[i_vmem.at[g]],
                    o_vmem.at[pl.ds(g * WINDOW, WINDOW)],
                )

        pltpu.emit_pipeline(
            body,
            grid=(num_ids // (GROUP * WINDOW),),
            in_specs=[
                pl.BlockSpec((GROUP, WINDOW), index_map=lambda i: (i, 0))
            ],
            out_specs=[
                pl.BlockSpec(
                    (GROUP * WINDOW, EMBED_DIM), index_map=lambda i: (i, 0)
                )
            ],
            core_axis_name=("core", "subcore"),
            dimension_semantics=(pltpu.PARALLEL,),
        )(i_hbm, o_hbm)

    return gather_kernel(weight, ids_2d)


def kernel(input_ids, weight):
    batch, seq = input_ids.shape
    num_ids = batch * seq
    ids_2d = input_ids.reshape(num_ids // WINDOW, WINDOW).astype(jnp.int32)
    out = _gather_rows(weight, ids_2d, num_ids)
    return out.reshape(batch, seq, EMBED_DIM)


# speedup vs baseline: 7.8934x; 1.0187x over previous
import functools

import jax
import jax.numpy as jnp
from jax.experimental import pallas as pl
from jax.experimental.pallas import tpu as pltpu
from jax.experimental.pallas import tpu_sc as plsc

EMBED_DIM = 128
WINDOW = 400
GROUP = 1


def _gather_rows(weight, ids_2d, num_ids):
    mesh = plsc.VectorSubcoreMesh(
        core_axis_name="core", subcore_axis_name="subcore"
    )

    @functools.partial(
        pl.kernel,
        out_type=jax.ShapeDtypeStruct((num_ids, EMBED_DIM), jnp.float32),
        mesh=mesh,
    )
    def gather_kernel(w_hbm, i_hbm, o_hbm):
        def body(i_vmem, o_vmem):
            for g in range(GROUP):
                pltpu.sync_copy(
                    w_hbm.at[i_vmem.at[g]],
                    o_vmem.at[pl.ds(g * WINDOW, WINDOW)],
                )

        pltpu.emit_pipeline(
            body,
            grid=(num_ids // (GROUP * WINDOW),),
            in_specs=[
                pl.BlockSpec((GROUP, WINDOW), index_map=lambda i: (i, 0))
            ],
            out_specs=[
                pl.BlockSpec(
                    (GROUP * WINDOW, EMBED_DIM), index_map=lambda i: (i, 0)
                )
            ],
            core_axis_name=("core", "subcore"),
            dimension_semantics=(pltpu.PARALLEL,),
        )(i_hbm, o_hbm)

    return gather_kernel(weight, ids_2d)


def kernel(input_ids, weight):
    batch, seq = input_ids.shape
    num_ids = batch * seq
    ids_2d = input_ids.reshape(num_ids // WINDOW, WINDOW).astype(jnp.int32)
    out = _gather_rows(weight, ids_2d, num_ids)
    return out.reshape(batch, seq, EMBED_DIM)


# speedup vs baseline: 8.0213x; 1.0162x over previous
import functools

import jax
import jax.numpy as jnp
from jax import lax
from jax.experimental import pallas as pl
from jax.experimental.pallas import tpu as pltpu
from jax.experimental.pallas import tpu_sc as plsc

EMBED_DIM = 128
CH = 400
NCH = 16
NBUF = 2
NUM_WORKERS = 32


def _gather_rows(weight, ids_2d, num_ids):
    rows_per_w = num_ids // NUM_WORKERS
    mesh = plsc.VectorSubcoreMesh(
        core_axis_name="core", subcore_axis_name="subcore"
    )

    @functools.partial(
        pl.kernel,
        out_type=jax.ShapeDtypeStruct((num_ids, EMBED_DIM), jnp.float32),
        mesh=mesh,
        scratch_types=[
            pltpu.VMEM((NCH * CH,), jnp.int32),
            pltpu.VMEM((CH, EMBED_DIM), jnp.float32),
            pltpu.VMEM((CH, EMBED_DIM), jnp.float32),
            pltpu.SemaphoreType.DMA,
            pltpu.SemaphoreType.DMA,
            pltpu.SemaphoreType.DMA,
            pltpu.SemaphoreType.DMA,
            pltpu.SemaphoreType.DMA,
        ],
    )
    def gather_kernel(
        w_hbm, i_hbm, o_hbm, idx_v, buf0, buf1, sem_i, sg0, sg1, sw0, sw1
    ):
        wid = lax.axis_index("subcore") * 2 + lax.axis_index("core")
        bufs = (buf0, buf1)
        sem_g = (sg0, sg1)
        sem_w = (sw0, sw1)

        pltpu.async_copy(i_hbm.at[wid], idx_v, sem_i).wait()

        base = wid * rows_per_w

        def gather_copy(g, b):
            return pltpu.make_async_copy(
                w_hbm.at[idx_v.at[pl.ds(g * CH, CH)]], bufs[b], sem_g[b]
            )

        def write_copy(g, b):
            return pltpu.make_async_copy(
                bufs[b], o_hbm.at[pl.ds(base + g * CH, CH)], sem_w[b]
            )

        for b in range(NBUF):
            gather_copy(b, b).start()

        for g in range(NCH - NBUF):
            b = g % NBUF
            gather_copy(g, b).wait()
            write_copy(g, b).start()
            write_copy(g, b).wait()
            gather_copy(g + NBUF, b).start()

        for b in range(NBUF):
            g = NCH - NBUF + b
            gather_copy(g, b).wait()
            write_copy(g, b).start()
        for b in range(NBUF):
            write_copy(NCH - NBUF + b, b).wait()

    return gather_kernel(weight, ids_2d)


def kernel(input_ids, weight):
    batch, seq = input_ids.shape
    num_ids = batch * seq
    ids_2d = input_ids.reshape(NUM_WORKERS, num_ids // NUM_WORKERS).astype(jnp.int32)
    out = _gather_rows(weight, ids_2d, num_ids)
    return out.reshape(batch, seq, EMBED_DIM)
